# Initial kernel scaffold; baseline (speedup 1.0000x reference)
import jax
import jax.numpy as jnp
from jax.experimental import pallas as pl


def kernel(xyz, points, xyz1, feats1_in, xyz0, feats0_in, params):
    raise NotImplementedError("write your pallas kernel here")


# speedup vs baseline: 17.7346x; 17.7346x over previous
import functools

import jax
import jax.numpy as jnp
from jax import lax
from jax.experimental import pallas as pl
from jax.experimental.pallas import tpu as pltpu
from jax.experimental.pallas import tpu_sc as plsc

_HI = lax.Precision.HIGHEST


def _dotT(x, w, precision=lax.Precision.DEFAULT):
    return lax.dot_general(x, w, (((1,), (1,)), ((), ())), precision=precision)


def _lin_bn_relu_body(x_ref, w_ref, b_ref, g_ref, bb_ref, o_ref):
    y = _dotT(x_ref[...], w_ref[...]) + b_ref[...]
    mean = jnp.mean(y, axis=0, keepdims=True)
    var = jnp.mean((y - mean) ** 2, axis=0, keepdims=True)
    yn = g_ref[...] * (y - mean) / jnp.sqrt(var + 1e-5) + bb_ref[...]
    o_ref[...] = jnp.maximum(yn, 0.0)


def _lin_bn_relu(x, w, b, g, bb):
    m = x.shape[0]
    dout = w.shape[0]
    return pl.pallas_call(
        _lin_bn_relu_body,
        out_shape=jax.ShapeDtypeStruct((m, dout), jnp.float32),
    )(x, w, b.reshape(1, -1), g.reshape(1, -1), bb.reshape(1, -1))


def _sq_dist(q, pT):
    e = lax.dot(q, pT, precision=lax.Precision.DEFAULT)
    qn = q[:, 0:1] ** 2 + q[:, 1:2] ** 2 + q[:, 2:3] ** 2
    pn = pT[0:1, :] ** 2 + pT[1:2, :] ** 2 + pT[2:3, :] ** 2
    return -2.0 * e + qn + pn


def _tu_interp_body(n_src, q_ref, pT_ref, f1_ref, f2_ref, o_ref):
    q = q_ref[...]
    pT = pT_ref[0]
    d = _sq_dist(q, pT)
    iota = lax.broadcasted_iota(jnp.int32, d.shape, 1)
    idxs, dists = [], []
    for _ in range(3):
        m = jnp.min(d, axis=1, keepdims=True)
        iv = jnp.min(jnp.where(d == m, iota, n_src), axis=1, keepdims=True)
        idxs.append(iv)
        dists.append(m)
        d = jnp.where(iota == iv, jnp.inf, d)
    recip = [1.0 / (dd + 1e-8) for dd in dists]
    norm = recip[0] + recip[1] + recip[2]
    wmat = jnp.zeros_like(d)
    for iv, r in zip(idxs, recip):
        wmat = wmat + jnp.where(iota == iv, r / norm, 0.0)
    o_ref[...] = lax.dot(wmat, f1_ref[0], precision=_HI) + f2_ref[...]


def _tu_interp(q_xyz_flat, src_xyzT, f1, f2, bsz, n_q, n_src, dout, blk):
    grid = (bsz, n_q // blk)
    return pl.pallas_call(
        functools.partial(_tu_interp_body, n_src),
        grid=grid,
        in_specs=[
            pl.BlockSpec((blk, 3), lambda b, n: (b * (n_q // blk) + n, 0)),
            pl.BlockSpec((1, 3, n_src), lambda b, n: (b, 0, 0)),
            pl.BlockSpec((1, n_src, dout), lambda b, n: (b, 0, 0)),
            pl.BlockSpec((blk, dout), lambda b, n: (b * (n_q // blk) + n, 0)),
        ],
        out_specs=pl.BlockSpec((blk, dout), lambda b, n: (b * (n_q // blk) + n, 0)),
        out_shape=jax.ShapeDtypeStruct((bsz * n_q, dout), jnp.float32),
    )(q_xyz_flat, src_xyzT, f1, f2)


def _tf_proj_body(f_ref, xyz_ref, fc1_ref, fc1b_ref, wq_ref, wk_ref, wv_ref,
                  d1_ref, q_ref, tab_ref):
    x = _dotT(f_ref[...], fc1_ref[...]) + fc1b_ref[...]
    q_ref[...] = _dotT(x, wq_ref[...])
    tab_ref[:, 0:64] = _dotT(x, wk_ref[...])
    tab_ref[:, 64:128] = _dotT(x, wv_ref[...])
    tab_ref[:, 128:192] = _dotT(xyz_ref[...], d1_ref[...], precision=_HI)
    tab_ref[:, 192:256] = jnp.zeros_like(q_ref[...])


def _tf_proj(feats, xyz_flat, p):
    m = feats.shape[0]
    dm = p['wq'].shape[0]
    return pl.pallas_call(
        _tf_proj_body,
        out_shape=[
            jax.ShapeDtypeStruct((m, dm), jnp.float32),
            jax.ShapeDtypeStruct((m, 256), jnp.float32),
        ],
    )(feats, xyz_flat, p['fc1_w'], p['fc1_b'].reshape(1, -1),
      p['wq'], p['wk'], p['wv'], p['d1_w'])


def _topk_body(n_pts, n_nb, q_ref, pT_ref, o_ref):
    b = pl.program_id(0)
    q = q_ref[...]
    pT = pT_ref[0]
    d = _sq_dist(q, pT)
    iota = lax.broadcasted_iota(jnp.int32, d.shape, 1)
    iota_nb = lax.broadcasted_iota(jnp.int32, (q.shape[0], n_nb), 1)
    acc = jnp.zeros((q.shape[0], n_nb), jnp.int32)
    for t in range(n_nb):
        m = jnp.min(d, axis=1, keepdims=True)
        iv = jnp.min(jnp.where(d == m, iota, n_pts), axis=1, keepdims=True)
        acc = jnp.where(iota_nb == t, iv, acc)
        d = jnp.where(iota == iv, jnp.inf, d)
    o_ref[...] = acc + b * n_pts


def _tf_topk(xyz_flat, xyzT, bsz, n_pts, n_nb, blk):
    grid = (bsz, n_pts // blk)
    return pl.pallas_call(
        functools.partial(_topk_body, n_pts, n_nb),
        grid=grid,
        in_specs=[
            pl.BlockSpec((blk, 3), lambda b, n: (b * (n_pts // blk) + n, 0)),
            pl.BlockSpec((1, 3, n_pts), lambda b, n: (b, 0, 0)),
        ],
        out_specs=pl.BlockSpec((blk, n_nb), lambda b, n: (b * (n_pts // blk) + n, 0)),
        out_shape=jax.ShapeDtypeStruct((bsz * n_pts, n_nb), jnp.int32),
    )(xyz_flat, xyzT)


def _make_sc_gather(n_idx, d):
    info = plsc.get_sparse_core_info()
    nw = info.num_cores * info.num_subcores
    per_w = n_idx // nw
    chunk = min(per_w, 256)
    n_ch = per_w // chunk
    assert per_w % chunk == 0 and n_idx % nw == 0 and d % 16 == 0
    mesh = plsc.VectorSubcoreMesh(core_axis_name="c", subcore_axis_name="s")

    @functools.partial(
        pl.kernel, mesh=mesh,
        out_type=jax.ShapeDtypeStruct((n_idx, d), jnp.float32),
        scratch_types=[
            pltpu.VMEM((chunk,), jnp.int32),
            pltpu.VMEM((chunk, d), jnp.float32),
            pltpu.SemaphoreType.DMA,
        ],
    )
    def gather(table_hbm, idx_hbm, out_hbm, idx_v, rows_v, sem):
        wid = lax.axis_index("s") * info.num_cores + lax.axis_index("c")
        base = wid * per_w

        def body(c, carry):
            off = base + c * chunk
            pltpu.sync_copy(idx_hbm.at[pl.ds(off, chunk)], idx_v)
            pltpu.async_copy(table_hbm.at[idx_v], rows_v, sem).wait()
            pltpu.sync_copy(rows_v, out_hbm.at[pl.ds(off, chunk)])
            return carry

        lax.fori_loop(0, n_ch, body, 0)

    return gather


def _tf_attn_body(g_ref, q_ref, px_ref, pre_ref,
                  d1b_ref, d2_ref, d2b_ref,
                  g1_ref, g1b_ref, g2_ref, g2b_ref,
                  fc2_ref, fc2b_ref, o_ref):
    q = q_ref[...]
    pxq = px_ref[...] + d1b_ref[...]
    d2 = d2_ref[...]
    g1 = g1_ref[...]
    g2 = g2_ref[...]
    pes, attns, vs = [], [], []
    for j in range(16):
        gj = g_ref[j]
        kj = gj[:, 0:64]
        vj = gj[:, 64:128]
        pe = _dotT(jnp.maximum(pxq - gj[:, 128:192], 0.0), d2) + d2b_ref[...]
        a = _dotT(jnp.maximum(_dotT(q - kj + pe, g1) + g1b_ref[...], 0.0), g2) \
            + g2b_ref[...]
        pes.append(pe)
        attns.append(a * 0.125)
        vs.append(vj)
    mx = attns[0]
    for a in attns[1:]:
        mx = jnp.maximum(mx, a)
    es = [jnp.exp(a - mx) for a in attns]
    tot = es[0]
    for e in es[1:]:
        tot = tot + e
    res = jnp.zeros_like(q)
    for e, pe, vj in zip(es, pes, vs):
        res = res + e * (vj + pe)
    res = res / tot
    o_ref[...] = _dotT(res, fc2_ref[...]) + fc2b_ref[...] + pre_ref[...]


def _tf_attn(gathered, qarr, px, pre, p, blk):
    m, dp = pre.shape
    grid = (m // blk,)
    full = lambda shape: pl.BlockSpec(shape, lambda n: (0,) * len(shape))
    return pl.pallas_call(
        _tf_attn_body,
        grid=grid,
        in_specs=[
            pl.BlockSpec((16, blk, 256), lambda n: (0, n, 0)),
            pl.BlockSpec((blk, 64), lambda n: (n, 0)),
            pl.BlockSpec((blk, 64), lambda n: (n, 0)),
            pl.BlockSpec((blk, dp), lambda n: (n, 0)),
            full((1, 64)), full((64, 64)), full((1, 64)),
            full((64, 64)), full((1, 64)), full((64, 64)), full((1, 64)),
            full((dp, 64)), full((1, dp)),
        ],
        out_specs=pl.BlockSpec((blk, dp), lambda n: (n, 0)),
        out_shape=jax.ShapeDtypeStruct((m, dp), jnp.float32),
    )(gathered, qarr, px, pre,
      p['d1_b'].reshape(1, -1), p['d2_w'], p['d2_b'].reshape(1, -1),
      p['g1_w'], p['g1_b'].reshape(1, -1), p['g2_w'], p['g2_b'].reshape(1, -1),
      p['fc2_w'], p['fc2_b'].reshape(1, -1))


def _transition_up(xyz_src, pts_src, xyz_q, pts_q, p, blk):
    bsz, n_src, d1 = pts_src.shape
    n_q, d2 = pts_q.shape[1], pts_q.shape[2]
    dout = p['fc1_w'].shape[0]
    f1 = _lin_bn_relu(pts_src.reshape(bsz * n_src, d1), p['fc1_w'], p['fc1_b'],
                      p['bn1_g'], p['bn1_b'])
    f2 = _lin_bn_relu(pts_q.reshape(bsz * n_q, d2), p['fc2_w'], p['fc2_b'],
                      p['bn2_g'], p['bn2_b'])
    src_T = jnp.transpose(xyz_src, (0, 2, 1))
    q_flat = xyz_q.reshape(bsz * n_q, 3)
    f1b = f1.reshape(bsz, n_src, dout)
    return _tu_interp(q_flat, src_T, f1b, f2, bsz, n_q, n_src, dout, blk)


def _transformer(xyz, feats_flat, p, blk_topk, blk_attn):
    bsz, n_pts = xyz.shape[:2]
    m = bsz * n_pts
    xyz_flat = xyz.reshape(m, 3)
    qarr, table = _tf_proj(feats_flat, xyz_flat, p)
    idx = _tf_topk(xyz_flat, jnp.transpose(xyz, (0, 2, 1)), bsz, n_pts, 16,
                   blk_topk)
    idx_nm = jnp.transpose(idx, (1, 0)).reshape(16 * m)
    gathered = _make_sc_gather(16 * m, 256)(table, idx_nm)
    gathered = gathered.reshape(16, m, 256)
    px = lax.slice(table, (0, 128), (m, 192))
    return _tf_attn(gathered, qarr, px, feats_flat, p, blk_attn)


def kernel(xyz, points, xyz1, feats1_in, xyz0, feats0_in, params):
    bsz, n2 = xyz.shape[:2]
    n1, n0 = xyz1.shape[1], xyz0.shape[1]

    pts = _transition_up(xyz, points, xyz1, feats1_in, params['tu0'], blk=256)
    pts = _transformer(xyz1, pts, params['tf0'], blk_topk=256, blk_attn=128)
    pts3 = pts.reshape(bsz, n1, -1)
    pts = _transition_up(xyz1, pts3, xyz0, feats0_in, params['tu1'], blk=256)
    pts = _transformer(xyz0, pts, params['tf1'], blk_topk=256, blk_attn=128)
    return xyz0, pts.reshape(bsz, n0, -1)


# speedup vs baseline: 21.9427x; 1.2373x over previous
import functools

import jax
import jax.numpy as jnp
from jax import lax
from jax.experimental import pallas as pl
from jax.experimental.pallas import tpu as pltpu
from jax.experimental.pallas import tpu_sc as plsc

_HI = lax.Precision.HIGHEST


def _dotT(x, w, precision=lax.Precision.DEFAULT):
    return lax.dot_general(x, w, (((1,), (1,)), ((), ())), precision=precision)


def _lin_bn_relu_body(x_ref, w_ref, b_ref, g_ref, bb_ref, o_ref):
    y = _dotT(x_ref[...], w_ref[...]) + b_ref[...]
    mean = jnp.mean(y, axis=0, keepdims=True)
    var = jnp.mean((y - mean) ** 2, axis=0, keepdims=True)
    yn = g_ref[...] * (y - mean) / jnp.sqrt(var + 1e-5) + bb_ref[...]
    o_ref[...] = jnp.maximum(yn, 0.0)


def _lin_bn_relu(x, w, b, g, bb):
    m = x.shape[0]
    dout = w.shape[0]
    return pl.pallas_call(
        _lin_bn_relu_body,
        out_shape=jax.ShapeDtypeStruct((m, dout), jnp.float32),
    )(x, w, b.reshape(1, -1), g.reshape(1, -1), bb.reshape(1, -1))


def _sq_dist(q, pT):
    e = lax.dot(q, pT, precision=lax.Precision.DEFAULT)
    qn = q[:, 0:1] ** 2 + q[:, 1:2] ** 2 + q[:, 2:3] ** 2
    pn = pT[0:1, :] ** 2 + pT[1:2, :] ** 2 + pT[2:3, :] ** 2
    return -2.0 * e + qn + pn


def _tu_interp_body(n_src, q_ref, pT_ref, f1_ref, f2_ref, o_ref):
    q = q_ref[...]
    pT = pT_ref[0]
    d = _sq_dist(q, pT)
    iota = lax.broadcasted_iota(jnp.int32, d.shape, 1)
    idxs, dists = [], []
    for _ in range(3):
        m = jnp.min(d, axis=1, keepdims=True)
        iv = jnp.min(jnp.where(d == m, iota, n_src), axis=1, keepdims=True)
        idxs.append(iv)
        dists.append(m)
        d = jnp.where(iota == iv, jnp.inf, d)
    recip = [1.0 / (dd + 1e-8) for dd in dists]
    norm = recip[0] + recip[1] + recip[2]
    wmat = jnp.zeros_like(d)
    for iv, r in zip(idxs, recip):
        wmat = wmat + jnp.where(iota == iv, r / norm, 0.0)
    o_ref[...] = lax.dot(wmat, f1_ref[0], precision=_HI) + f2_ref[...]


def _tu_interp(q_xyz_flat, src_xyzT, f1, f2, bsz, n_q, n_src, dout, blk):
    grid = (bsz, n_q // blk)
    return pl.pallas_call(
        functools.partial(_tu_interp_body, n_src),
        grid=grid,
        in_specs=[
            pl.BlockSpec((blk, 3), lambda b, n: (b * (n_q // blk) + n, 0)),
            pl.BlockSpec((1, 3, n_src), lambda b, n: (b, 0, 0)),
            pl.BlockSpec((1, n_src, dout), lambda b, n: (b, 0, 0)),
            pl.BlockSpec((blk, dout), lambda b, n: (b * (n_q // blk) + n, 0)),
        ],
        out_specs=pl.BlockSpec((blk, dout), lambda b, n: (b * (n_q // blk) + n, 0)),
        out_shape=jax.ShapeDtypeStruct((bsz * n_q, dout), jnp.float32),
    )(q_xyz_flat, src_xyzT, f1, f2)


def _tf_proj_body(f_ref, xyz_ref, fc1_ref, fc1b_ref, wq_ref, wk_ref, wv_ref,
                  d1_ref, q_ref, tab_ref):
    x = _dotT(f_ref[...], fc1_ref[...]) + fc1b_ref[...]
    q_ref[...] = _dotT(x, wq_ref[...])
    tab_ref[:, 0:64] = _dotT(x, wk_ref[...])
    tab_ref[:, 64:128] = _dotT(x, wv_ref[...])
    tab_ref[:, 128:192] = _dotT(xyz_ref[...], d1_ref[...], precision=_HI)
    tab_ref[:, 192:256] = jnp.zeros_like(q_ref[...])


def _tf_proj(feats, xyz_flat, p):
    m = feats.shape[0]
    dm = p['wq'].shape[0]
    return pl.pallas_call(
        _tf_proj_body,
        out_shape=[
            jax.ShapeDtypeStruct((m, dm), jnp.float32),
            jax.ShapeDtypeStruct((m, 256), jnp.float32),
        ],
    )(feats, xyz_flat, p['fc1_w'], p['fc1_b'].reshape(1, -1),
      p['wq'], p['wk'], p['wv'], p['d1_w'])


def _topk_body(n_pts, n_nb, q_ref, pT_ref, o_ref):
    b = pl.program_id(0)
    q = q_ref[...]
    pT = pT_ref[0]
    d = _sq_dist(q, pT)
    iota = lax.broadcasted_iota(jnp.int32, d.shape, 1)
    iota_nb = lax.broadcasted_iota(jnp.int32, (q.shape[0], n_nb), 1)
    acc = jnp.zeros((q.shape[0], n_nb), jnp.int32)
    for t in range(n_nb):
        iv = jnp.argmin(d, axis=1)[:, None]
        acc = jnp.where(iota_nb == t, iv, acc)
        d = jnp.where(iota == iv, jnp.inf, d)
    o_ref[...] = acc + b * n_pts


def _tf_topk(xyz_flat, xyzT, bsz, n_pts, n_nb, blk):
    grid = (bsz, n_pts // blk)
    return pl.pallas_call(
        functools.partial(_topk_body, n_pts, n_nb),
        grid=grid,
        in_specs=[
            pl.BlockSpec((blk, 3), lambda b, n: (b * (n_pts // blk) + n, 0)),
            pl.BlockSpec((1, 3, n_pts), lambda b, n: (b, 0, 0)),
        ],
        out_specs=pl.BlockSpec((blk, n_nb), lambda b, n: (b * (n_pts // blk) + n, 0)),
        out_shape=jax.ShapeDtypeStruct((bsz * n_pts, n_nb), jnp.int32),
    )(xyz_flat, xyzT)


def _make_sc_gather(n_idx, d):
    info = plsc.get_sparse_core_info()
    nw = info.num_cores * info.num_subcores
    per_w = n_idx // nw
    chunk = min(per_w, 128)
    n_pair = per_w // (2 * chunk)
    assert per_w % (2 * chunk) == 0 and n_idx % nw == 0 and d % 16 == 0
    mesh = plsc.VectorSubcoreMesh(core_axis_name="c", subcore_axis_name="s")

    @functools.partial(
        pl.kernel, mesh=mesh,
        out_type=jax.ShapeDtypeStruct((n_idx, d), jnp.float32),
        scratch_types=[
            pltpu.VMEM((per_w,), jnp.int32),
            pltpu.VMEM((chunk, d), jnp.float32),
            pltpu.VMEM((chunk, d), jnp.float32),
            pltpu.SemaphoreType.DMA,
            pltpu.SemaphoreType.DMA,
            pltpu.SemaphoreType.DMA,
            pltpu.SemaphoreType.DMA,
        ],
    )
    def gather(table_hbm, idx_hbm, out_hbm, idx_v, rows0, rows1,
               gsem0, gsem1, osem0, osem1):
        wid = lax.axis_index("s") * info.num_cores + lax.axis_index("c")
        base = wid * per_w
        pltpu.sync_copy(idx_hbm.at[pl.ds(base, per_w)], idx_v)

        def body(i, carry):
            c0 = 2 * i * chunk
            c1 = c0 + chunk
            g0 = pltpu.async_copy(
                table_hbm.at[idx_v.at[pl.ds(c0, chunk)]], rows0, gsem0)
            g1 = pltpu.async_copy(
                table_hbm.at[idx_v.at[pl.ds(c1, chunk)]], rows1, gsem1)
            g0.wait()
            o0 = pltpu.async_copy(rows0, out_hbm.at[pl.ds(base + c0, chunk)],
                                  osem0)
            g1.wait()
            o1 = pltpu.async_copy(rows1, out_hbm.at[pl.ds(base + c1, chunk)],
                                  osem1)
            o0.wait()
            o1.wait()
            return carry

        lax.fori_loop(0, n_pair, body, 0)

    return gather


def _tf_attn_body(g_ref, q_ref, px_ref, pre_ref,
                  d1b_ref, d2_ref, d2b_ref,
                  g1_ref, g1b_ref, g2_ref, g2b_ref,
                  fc2_ref, fc2b_ref, o_ref):
    t = q_ref.shape[0]
    q = q_ref[...]
    pxq = px_ref[...] + d1b_ref[...]
    d2 = d2_ref[...]
    g1 = g1_ref[...]
    g2 = g2_ref[...]
    g2d = g_ref[...].reshape(16 * t, 256)
    qs = jnp.broadcast_to(q[None], (16, t, 64)).reshape(16 * t, 64)
    pxqs = jnp.broadcast_to(pxq[None], (16, t, 64)).reshape(16 * t, 64)
    pe = _dotT(jnp.maximum(pxqs - g2d[:, 128:192], 0.0), d2) + d2b_ref[...]
    a = _dotT(jnp.maximum(_dotT(qs - g2d[:, 0:64] + pe, g1) + g1b_ref[...],
                          0.0), g2) + g2b_ref[...]
    a = a * 0.125
    vpe = g2d[:, 64:128] + pe
    attns = [a[j * t:(j + 1) * t] for j in range(16)]
    mx = attns[0]
    for aj in attns[1:]:
        mx = jnp.maximum(mx, aj)
    es = [jnp.exp(aj - mx) for aj in attns]
    tot = es[0]
    for e in es[1:]:
        tot = tot + e
    res = jnp.zeros_like(q)
    for j, e in enumerate(es):
        res = res + e * vpe[j * t:(j + 1) * t]
    res = res / tot
    o_ref[...] = _dotT(res, fc2_ref[...]) + fc2b_ref[...] + pre_ref[...]


def _tf_attn(gathered, qarr, px, pre, p, blk):
    m, dp = pre.shape
    grid = (m // blk,)
    full = lambda shape: pl.BlockSpec(shape, lambda n: (0,) * len(shape))
    return pl.pallas_call(
        _tf_attn_body,
        grid=grid,
        in_specs=[
            pl.BlockSpec((16, blk, 256), lambda n: (0, n, 0)),
            pl.BlockSpec((blk, 64), lambda n: (n, 0)),
            pl.BlockSpec((blk, 64), lambda n: (n, 0)),
            pl.BlockSpec((blk, dp), lambda n: (n, 0)),
            full((1, 64)), full((64, 64)), full((1, 64)),
            full((64, 64)), full((1, 64)), full((64, 64)), full((1, 64)),
            full((dp, 64)), full((1, dp)),
        ],
        out_specs=pl.BlockSpec((blk, dp), lambda n: (n, 0)),
        out_shape=jax.ShapeDtypeStruct((m, dp), jnp.float32),
    )(gathered, qarr, px, pre,
      p['d1_b'].reshape(1, -1), p['d2_w'], p['d2_b'].reshape(1, -1),
      p['g1_w'], p['g1_b'].reshape(1, -1), p['g2_w'], p['g2_b'].reshape(1, -1),
      p['fc2_w'], p['fc2_b'].reshape(1, -1))


def _transition_up(xyz_src, pts_src, xyz_q, pts_q, p, blk):
    bsz, n_src, d1 = pts_src.shape
    n_q, d2 = pts_q.shape[1], pts_q.shape[2]
    dout = p['fc1_w'].shape[0]
    f1 = _lin_bn_relu(pts_src.reshape(bsz * n_src, d1), p['fc1_w'], p['fc1_b'],
                      p['bn1_g'], p['bn1_b'])
    f2 = _lin_bn_relu(pts_q.reshape(bsz * n_q, d2), p['fc2_w'], p['fc2_b'],
                      p['bn2_g'], p['bn2_b'])
    src_T = jnp.transpose(xyz_src, (0, 2, 1))
    q_flat = xyz_q.reshape(bsz * n_q, 3)
    f1b = f1.reshape(bsz, n_src, dout)
    return _tu_interp(q_flat, src_T, f1b, f2, bsz, n_q, n_src, dout, blk)


def _transformer(xyz, feats_flat, p, blk_topk, blk_attn):
    bsz, n_pts = xyz.shape[:2]
    m = bsz * n_pts
    xyz_flat = xyz.reshape(m, 3)
    qarr, table = _tf_proj(feats_flat, xyz_flat, p)
    idx = _tf_topk(xyz_flat, jnp.transpose(xyz, (0, 2, 1)), bsz, n_pts, 16,
                   blk_topk)
    idx_nm = jnp.transpose(idx, (1, 0)).reshape(16 * m)
    gathered = _make_sc_gather(16 * m, 256)(table, idx_nm)
    gathered = gathered.reshape(16, m, 256)
    px = lax.slice(table, (0, 128), (m, 192))
    return _tf_attn(gathered, qarr, px, feats_flat, p, blk_attn)


def kernel(xyz, points, xyz1, feats1_in, xyz0, feats0_in, params):
    bsz, n2 = xyz.shape[:2]
    n1, n0 = xyz1.shape[1], xyz0.shape[1]

    pts = _transition_up(xyz, points, xyz1, feats1_in, params['tu0'], blk=256)
    pts = _transformer(xyz1, pts, params['tf0'], blk_topk=256, blk_attn=128)
    pts3 = pts.reshape(bsz, n1, -1)
    pts = _transition_up(xyz1, pts3, xyz0, feats0_in, params['tu1'], blk=256)
    pts = _transformer(xyz0, pts, params['tf1'], blk_topk=256, blk_attn=128)
    return xyz0, pts.reshape(bsz, n0, -1)


# speedup vs baseline: 22.6266x; 1.0312x over previous
import functools

import jax
import jax.numpy as jnp
from jax import lax
from jax.experimental import pallas as pl
from jax.experimental.pallas import tpu as pltpu
from jax.experimental.pallas import tpu_sc as plsc

_HI = lax.Precision.HIGHEST


def _dotT(x, w, precision=lax.Precision.DEFAULT):
    return lax.dot_general(x, w, (((1,), (1,)), ((), ())), precision=precision)


def _lin_bn_relu_body(x_ref, w_ref, b_ref, g_ref, bb_ref, o_ref):
    y = _dotT(x_ref[...], w_ref[...]) + b_ref[...]
    mean = jnp.mean(y, axis=0, keepdims=True)
    var = jnp.mean((y - mean) ** 2, axis=0, keepdims=True)
    yn = g_ref[...] * (y - mean) / jnp.sqrt(var + 1e-5) + bb_ref[...]
    o_ref[...] = jnp.maximum(yn, 0.0)


def _lin_bn_relu(x, w, b, g, bb):
    m = x.shape[0]
    dout = w.shape[0]
    return pl.pallas_call(
        _lin_bn_relu_body,
        out_shape=jax.ShapeDtypeStruct((m, dout), jnp.float32),
    )(x, w, b.reshape(1, -1), g.reshape(1, -1), bb.reshape(1, -1))


def _sq_dist(q, pT):
    e = lax.dot(q, pT, precision=lax.Precision.DEFAULT)
    qn = q[:, 0:1] ** 2 + q[:, 1:2] ** 2 + q[:, 2:3] ** 2
    pn = pT[0:1, :] ** 2 + pT[1:2, :] ** 2 + pT[2:3, :] ** 2
    return -2.0 * e + qn + pn


def _tu_interp_body(n_src, q_ref, pT_ref, f1_ref, f2_ref, o_ref):
    q = q_ref[...]
    pT = pT_ref[0]
    d = _sq_dist(q, pT)
    iota = lax.broadcasted_iota(jnp.int32, d.shape, 1)
    idxs, dists = [], []
    for _ in range(3):
        m = jnp.min(d, axis=1, keepdims=True)
        iv = jnp.min(jnp.where(d == m, iota, n_src), axis=1, keepdims=True)
        idxs.append(iv)
        dists.append(m)
        d = jnp.where(iota == iv, jnp.inf, d)
    recip = [1.0 / (dd + 1e-8) for dd in dists]
    norm = recip[0] + recip[1] + recip[2]
    wmat = jnp.zeros_like(d)
    for iv, r in zip(idxs, recip):
        wmat = wmat + jnp.where(iota == iv, r / norm, 0.0)
    o_ref[...] = lax.dot(wmat, f1_ref[0], precision=_HI) + f2_ref[...]


def _tu_interp(q_xyz_flat, src_xyzT, f1, f2, bsz, n_q, n_src, dout, blk):
    grid = (bsz, n_q // blk)
    return pl.pallas_call(
        functools.partial(_tu_interp_body, n_src),
        grid=grid,
        in_specs=[
            pl.BlockSpec((blk, 3), lambda b, n: (b * (n_q // blk) + n, 0)),
            pl.BlockSpec((1, 3, n_src), lambda b, n: (b, 0, 0)),
            pl.BlockSpec((1, n_src, dout), lambda b, n: (b, 0, 0)),
            pl.BlockSpec((blk, dout), lambda b, n: (b * (n_q // blk) + n, 0)),
        ],
        out_specs=pl.BlockSpec((blk, dout), lambda b, n: (b * (n_q // blk) + n, 0)),
        out_shape=jax.ShapeDtypeStruct((bsz * n_q, dout), jnp.float32),
    )(q_xyz_flat, src_xyzT, f1, f2)


def _tf_proj_body(f_ref, xyz_ref, fc1_ref, fc1b_ref, wq_ref, wk_ref, wv_ref,
                  d1_ref, q_ref, tab_ref):
    x = _dotT(f_ref[...], fc1_ref[...]) + fc1b_ref[...]
    q_ref[...] = _dotT(x, wq_ref[...])
    tab_ref[:, 0:64] = _dotT(x, wk_ref[...])
    tab_ref[:, 64:128] = _dotT(x, wv_ref[...])
    tab_ref[:, 128:192] = _dotT(xyz_ref[...], d1_ref[...], precision=_HI)
    tab_ref[:, 192:256] = jnp.zeros_like(q_ref[...])


def _tf_proj(feats, xyz_flat, p):
    m = feats.shape[0]
    dm = p['wq'].shape[0]
    return pl.pallas_call(
        _tf_proj_body,
        out_shape=[
            jax.ShapeDtypeStruct((m, dm), jnp.float32),
            jax.ShapeDtypeStruct((m, 256), jnp.float32),
        ],
    )(feats, xyz_flat, p['fc1_w'], p['fc1_b'].reshape(1, -1),
      p['wq'], p['wk'], p['wv'], p['d1_w'])


def _topk_body(n_pts, n_nb, base, q_ref, pT_ref, o_ref):
    q = q_ref[...]
    pT = pT_ref[0]
    d = _sq_dist(q, pT)
    iota = lax.broadcasted_iota(jnp.int32, d.shape, 1)
    iota_nb = lax.broadcasted_iota(jnp.int32, (q.shape[0], n_nb), 1)
    acc = jnp.zeros((q.shape[0], n_nb), jnp.int32)
    for t in range(n_nb):
        iv = jnp.argmin(d, axis=1)[:, None]
        acc = jnp.where(iota_nb == t, iv, acc)
        d = jnp.where(iota == iv, jnp.inf, d)
    o_ref[...] = acc + base


def _tf_topk(xyz_flat_b, xyzT_b, n_pts, n_nb, blk, base):
    return pl.pallas_call(
        functools.partial(_topk_body, n_pts, n_nb, base),
        grid=(n_pts // blk,),
        in_specs=[
            pl.BlockSpec((blk, 3), lambda n: (n, 0)),
            pl.BlockSpec((1, 3, n_pts), lambda n: (0, 0, 0)),
        ],
        out_specs=pl.BlockSpec((blk, n_nb), lambda n: (n, 0)),
        out_shape=jax.ShapeDtypeStruct((n_pts, n_nb), jnp.int32),
    )(xyz_flat_b, xyzT_b)


def _make_sc_gather(n_idx, d):
    info = plsc.get_sparse_core_info()
    nw = info.num_cores * info.num_subcores
    per_w = n_idx // nw
    chunk = min(per_w, 128)
    n_pair = per_w // (2 * chunk)
    assert per_w % (2 * chunk) == 0 and n_idx % nw == 0 and d % 16 == 0
    mesh = plsc.VectorSubcoreMesh(core_axis_name="c", subcore_axis_name="s")

    @functools.partial(
        pl.kernel, mesh=mesh,
        out_type=jax.ShapeDtypeStruct((n_idx, d), jnp.float32),
        scratch_types=[
            pltpu.VMEM((per_w,), jnp.int32),
            pltpu.VMEM((chunk, d), jnp.float32),
            pltpu.VMEM((chunk, d), jnp.float32),
            pltpu.SemaphoreType.DMA,
            pltpu.SemaphoreType.DMA,
            pltpu.SemaphoreType.DMA,
            pltpu.SemaphoreType.DMA,
        ],
    )
    def gather(table_hbm, idx_hbm, out_hbm, idx_v, rows0, rows1,
               gsem0, gsem1, osem0, osem1):
        wid = lax.axis_index("s") * info.num_cores + lax.axis_index("c")
        base = wid * per_w
        pltpu.sync_copy(idx_hbm.at[pl.ds(base, per_w)], idx_v)

        def body(i, carry):
            c0 = 2 * i * chunk
            c1 = c0 + chunk
            g0 = pltpu.async_copy(
                table_hbm.at[idx_v.at[pl.ds(c0, chunk)]], rows0, gsem0)
            g1 = pltpu.async_copy(
                table_hbm.at[idx_v.at[pl.ds(c1, chunk)]], rows1, gsem1)
            g0.wait()
            o0 = pltpu.async_copy(rows0, out_hbm.at[pl.ds(base + c0, chunk)],
                                  osem0)
            g1.wait()
            o1 = pltpu.async_copy(rows1, out_hbm.at[pl.ds(base + c1, chunk)],
                                  osem1)
            o0.wait()
            o1.wait()
            return carry

        lax.fori_loop(0, n_pair, body, 0)

    return gather


def _tf_attn_body(g_ref, q_ref, px_ref, pre_ref,
                  d1b_ref, d2_ref, d2b_ref,
                  g1_ref, g1b_ref, g2_ref, g2b_ref,
                  fc2_ref, fc2b_ref, o_ref):
    t = q_ref.shape[0]
    q = q_ref[...]
    pxq = px_ref[...] + d1b_ref[...]
    d2 = d2_ref[...]
    g1 = g1_ref[...]
    g2 = g2_ref[...]
    g2d = g_ref[...].reshape(16 * t, 256)
    qs = jnp.broadcast_to(q[None], (16, t, 64)).reshape(16 * t, 64)
    pxqs = jnp.broadcast_to(pxq[None], (16, t, 64)).reshape(16 * t, 64)
    pe = _dotT(jnp.maximum(pxqs - g2d[:, 128:192], 0.0), d2) + d2b_ref[...]
    a = _dotT(jnp.maximum(_dotT(qs - g2d[:, 0:64] + pe, g1) + g1b_ref[...],
                          0.0), g2) + g2b_ref[...]
    a = a * 0.125
    vpe = g2d[:, 64:128] + pe
    attns = [a[j * t:(j + 1) * t] for j in range(16)]
    mx = attns[0]
    for aj in attns[1:]:
        mx = jnp.maximum(mx, aj)
    es = [jnp.exp(aj - mx) for aj in attns]
    tot = es[0]
    for e in es[1:]:
        tot = tot + e
    res = jnp.zeros_like(q)
    for j, e in enumerate(es):
        res = res + e * vpe[j * t:(j + 1) * t]
    res = res / tot
    o_ref[...] = _dotT(res, fc2_ref[...]) + fc2b_ref[...] + pre_ref[...]


def _tf_attn(gathered, qarr, px, pre, p, blk):
    m, dp = pre.shape
    grid = (m // blk,)
    full = lambda shape: pl.BlockSpec(shape, lambda n: (0,) * len(shape))
    return pl.pallas_call(
        _tf_attn_body,
        grid=grid,
        in_specs=[
            pl.BlockSpec((16, blk, 256), lambda n: (0, n, 0)),
            pl.BlockSpec((blk, 64), lambda n: (n, 0)),
            pl.BlockSpec((blk, 64), lambda n: (n, 0)),
            pl.BlockSpec((blk, dp), lambda n: (n, 0)),
            full((1, 64)), full((64, 64)), full((1, 64)),
            full((64, 64)), full((1, 64)), full((64, 64)), full((1, 64)),
            full((dp, 64)), full((1, dp)),
        ],
        out_specs=pl.BlockSpec((blk, dp), lambda n: (n, 0)),
        out_shape=jax.ShapeDtypeStruct((m, dp), jnp.float32),
    )(gathered, qarr, px, pre,
      p['d1_b'].reshape(1, -1), p['d2_w'], p['d2_b'].reshape(1, -1),
      p['g1_w'], p['g1_b'].reshape(1, -1), p['g2_w'], p['g2_b'].reshape(1, -1),
      p['fc2_w'], p['fc2_b'].reshape(1, -1))


def _transition_up(xyz_src, pts_src, xyz_q, pts_q, p, blk):
    bsz, n_src, d1 = pts_src.shape
    n_q, d2 = pts_q.shape[1], pts_q.shape[2]
    dout = p['fc1_w'].shape[0]
    f1 = _lin_bn_relu(pts_src.reshape(bsz * n_src, d1), p['fc1_w'], p['fc1_b'],
                      p['bn1_g'], p['bn1_b'])
    f2 = _lin_bn_relu(pts_q.reshape(bsz * n_q, d2), p['fc2_w'], p['fc2_b'],
                      p['bn2_g'], p['bn2_b'])
    src_T = jnp.transpose(xyz_src, (0, 2, 1))
    q_flat = xyz_q.reshape(bsz * n_q, 3)
    f1b = f1.reshape(bsz, n_src, dout)
    return _tu_interp(q_flat, src_T, f1b, f2, bsz, n_q, n_src, dout, blk)


def _transformer(xyz, feats_flat, p, blk_topk, blk_attn):
    bsz, n_pts = xyz.shape[:2]
    m = bsz * n_pts
    xyz_flat = xyz.reshape(m, 3)
    xyzT = jnp.transpose(xyz, (0, 2, 1))
    qarr, table = _tf_proj(feats_flat, xyz_flat, p)
    sc_gather = _make_sc_gather(16 * n_pts, 256)
    idxs = [_tf_topk(xyz_flat[b * n_pts:(b + 1) * n_pts], xyzT[b:b + 1],
                     n_pts, 16, blk_topk, b * n_pts) for b in range(bsz)]
    gathers = [sc_gather(table, jnp.transpose(i, (1, 0)).reshape(16 * n_pts))
               for i in idxs]
    outs = []
    for b in range(bsz):
        lo, hi = b * n_pts, (b + 1) * n_pts
        outs.append(_tf_attn(gathers[b].reshape(16, n_pts, 256),
                             qarr[lo:hi], lax.slice(table, (lo, 128), (hi, 192)),
                             feats_flat[lo:hi], p, blk_attn))
    return jnp.concatenate(outs, axis=0)


def kernel(xyz, points, xyz1, feats1_in, xyz0, feats0_in, params):
    bsz, n2 = xyz.shape[:2]
    n1, n0 = xyz1.shape[1], xyz0.shape[1]

    pts = _transition_up(xyz, points, xyz1, feats1_in, params['tu0'], blk=256)
    pts = _transformer(xyz1, pts, params['tf0'], blk_topk=256, blk_attn=128)
    pts3 = pts.reshape(bsz, n1, -1)
    pts = _transition_up(xyz1, pts3, xyz0, feats0_in, params['tu1'], blk=256)
    pts = _transformer(xyz0, pts, params['tf1'], blk_topk=256, blk_attn=128)
    return xyz0, pts.reshape(bsz, n0, -1)


# speedup vs baseline: 23.5484x; 1.0407x over previous
import functools

import jax
import jax.numpy as jnp
from jax import lax
from jax.experimental import pallas as pl
from jax.experimental.pallas import tpu as pltpu
from jax.experimental.pallas import tpu_sc as plsc

_HI = lax.Precision.HIGHEST


def _dotT(x, w, precision=lax.Precision.DEFAULT):
    return lax.dot_general(x, w, (((1,), (1,)), ((), ())), precision=precision)


def _lin_bn_relu_body(x_ref, w_ref, b_ref, g_ref, bb_ref, o_ref):
    y = _dotT(x_ref[...], w_ref[...]) + b_ref[...]
    mean = jnp.mean(y, axis=0, keepdims=True)
    var = jnp.mean((y - mean) ** 2, axis=0, keepdims=True)
    yn = g_ref[...] * (y - mean) / jnp.sqrt(var + 1e-5) + bb_ref[...]
    o_ref[...] = jnp.maximum(yn, 0.0)


def _lin_bn_relu(x, w, b, g, bb):
    m = x.shape[0]
    dout = w.shape[0]
    return pl.pallas_call(
        _lin_bn_relu_body,
        out_shape=jax.ShapeDtypeStruct((m, dout), jnp.float32),
    )(x, w, b.reshape(1, -1), g.reshape(1, -1), bb.reshape(1, -1))


def _sq_dist(q, pT):
    e = lax.dot(q, pT, precision=lax.Precision.DEFAULT)
    qn = q[:, 0:1] ** 2 + q[:, 1:2] ** 2 + q[:, 2:3] ** 2
    pn = pT[0:1, :] ** 2 + pT[1:2, :] ** 2 + pT[2:3, :] ** 2
    return -2.0 * e + qn + pn


def _tu_interp_body(n_src, q_ref, pT_ref, f1_ref, f2_ref, o_ref):
    q = q_ref[...]
    pT = pT_ref[0]
    d = _sq_dist(q, pT)
    iota = lax.broadcasted_iota(jnp.int32, d.shape, 1)
    idxs, dists = [], []
    for _ in range(3):
        m = jnp.min(d, axis=1, keepdims=True)
        iv = jnp.min(jnp.where(d == m, iota, n_src), axis=1, keepdims=True)
        idxs.append(iv)
        dists.append(m)
        d = jnp.where(iota == iv, jnp.inf, d)
    recip = [1.0 / (dd + 1e-8) for dd in dists]
    norm = recip[0] + recip[1] + recip[2]
    wmat = jnp.zeros_like(d)
    for iv, r in zip(idxs, recip):
        wmat = wmat + jnp.where(iota == iv, r / norm, 0.0)
    o_ref[...] = lax.dot(wmat, f1_ref[0], precision=_HI) + f2_ref[...]


def _tu_interp(q_xyz_flat, src_xyzT, f1, f2, bsz, n_q, n_src, dout, blk):
    grid = (bsz, n_q // blk)
    return pl.pallas_call(
        functools.partial(_tu_interp_body, n_src),
        grid=grid,
        in_specs=[
            pl.BlockSpec((blk, 3), lambda b, n: (b * (n_q // blk) + n, 0)),
            pl.BlockSpec((1, 3, n_src), lambda b, n: (b, 0, 0)),
            pl.BlockSpec((1, n_src, dout), lambda b, n: (b, 0, 0)),
            pl.BlockSpec((blk, dout), lambda b, n: (b * (n_q // blk) + n, 0)),
        ],
        out_specs=pl.BlockSpec((blk, dout), lambda b, n: (b * (n_q // blk) + n, 0)),
        out_shape=jax.ShapeDtypeStruct((bsz * n_q, dout), jnp.float32),
    )(q_xyz_flat, src_xyzT, f1, f2)


def _rne_hi16(x):
    return (x + 0x7FFF + ((x >> 16) & 1)) & jnp.int32(-65536)


def _tf_proj_body(f_ref, xyz_ref, fc1_ref, fc1b_ref, wq_ref, wk_ref, wv_ref,
                  d1_ref, q_ref, tab_ref):
    x = _dotT(f_ref[...], fc1_ref[...]) + fc1b_ref[...]
    q_ref[...] = _dotT(x, wq_ref[...])
    ki = lax.bitcast_convert_type(_dotT(x, wk_ref[...]), jnp.int32)
    vi = lax.bitcast_convert_type(_dotT(x, wv_ref[...]), jnp.int32)
    kv = _rne_hi16(ki) | ((_rne_hi16(vi) >> 16) & 0xFFFF)
    tab_ref[:, 0:64] = lax.bitcast_convert_type(kv, jnp.float32)
    tab_ref[:, 64:128] = _dotT(xyz_ref[...], d1_ref[...], precision=_HI)


def _tf_proj(feats, xyz_flat, p):
    m = feats.shape[0]
    dm = p['wq'].shape[0]
    return pl.pallas_call(
        _tf_proj_body,
        out_shape=[
            jax.ShapeDtypeStruct((m, dm), jnp.float32),
            jax.ShapeDtypeStruct((m, 128), jnp.float32),
        ],
    )(feats, xyz_flat, p['fc1_w'], p['fc1_b'].reshape(1, -1),
      p['wq'], p['wk'], p['wv'], p['d1_w'])


def _topk_body(n_pts, n_nb, base, q_ref, pT_ref, o_ref):
    q = q_ref[...]
    pT = pT_ref[0]
    d = _sq_dist(q, pT)
    iota = lax.broadcasted_iota(jnp.int32, d.shape, 1)
    iota_nb = lax.broadcasted_iota(jnp.int32, (q.shape[0], n_nb), 1)
    acc = jnp.zeros((q.shape[0], n_nb), jnp.int32)
    for t in range(n_nb):
        iv = jnp.argmin(d, axis=1)[:, None]
        acc = jnp.where(iota_nb == t, iv, acc)
        d = jnp.where(iota == iv, jnp.inf, d)
    o_ref[...] = acc + base


def _tf_topk(xyz_flat_b, xyzT_b, n_pts, n_nb, blk, base):
    return pl.pallas_call(
        functools.partial(_topk_body, n_pts, n_nb, base),
        grid=(n_pts // blk,),
        in_specs=[
            pl.BlockSpec((blk, 3), lambda n: (n, 0)),
            pl.BlockSpec((1, 3, n_pts), lambda n: (0, 0, 0)),
        ],
        out_specs=pl.BlockSpec((blk, n_nb), lambda n: (n, 0)),
        out_shape=jax.ShapeDtypeStruct((n_pts, n_nb), jnp.int32),
    )(xyz_flat_b, xyzT_b)


def _make_sc_gather(n_idx, d):
    info = plsc.get_sparse_core_info()
    nw = info.num_cores * info.num_subcores
    per_w = n_idx // nw
    chunk = min(per_w, 256)
    n_pair = per_w // (2 * chunk)
    assert per_w % (2 * chunk) == 0 and n_idx % nw == 0 and d % 16 == 0
    mesh = plsc.VectorSubcoreMesh(core_axis_name="c", subcore_axis_name="s")

    @functools.partial(
        pl.kernel, mesh=mesh,
        out_type=jax.ShapeDtypeStruct((n_idx, d), jnp.float32),
        scratch_types=[
            pltpu.VMEM((per_w,), jnp.int32),
            pltpu.VMEM((chunk, d), jnp.float32),
            pltpu.VMEM((chunk, d), jnp.float32),
            pltpu.SemaphoreType.DMA,
            pltpu.SemaphoreType.DMA,
            pltpu.SemaphoreType.DMA,
            pltpu.SemaphoreType.DMA,
        ],
    )
    def gather(table_hbm, idx_hbm, out_hbm, idx_v, rows0, rows1,
               gsem0, gsem1, osem0, osem1):
        wid = lax.axis_index("s") * info.num_cores + lax.axis_index("c")
        base = wid * per_w
        pltpu.sync_copy(idx_hbm.at[pl.ds(base, per_w)], idx_v)

        def body(i, carry):
            c0 = 2 * i * chunk
            c1 = c0 + chunk
            g0 = pltpu.async_copy(
                table_hbm.at[idx_v.at[pl.ds(c0, chunk)]], rows0, gsem0)
            g1 = pltpu.async_copy(
                table_hbm.at[idx_v.at[pl.ds(c1, chunk)]], rows1, gsem1)
            g0.wait()
            o0 = pltpu.async_copy(rows0, out_hbm.at[pl.ds(base + c0, chunk)],
                                  osem0)
            g1.wait()
            o1 = pltpu.async_copy(rows1, out_hbm.at[pl.ds(base + c1, chunk)],
                                  osem1)
            o0.wait()
            o1.wait()
            return carry

        lax.fori_loop(0, n_pair, body, 0)

    return gather


def _tf_attn_body(g_ref, q_ref, px_ref, pre_ref,
                  d1b_ref, d2_ref, d2b_ref,
                  g1_ref, g1b_ref, g2_ref, g2b_ref,
                  fc2_ref, fc2b_ref, o_ref):
    t = q_ref.shape[0]
    q = q_ref[...]
    pxq = px_ref[...] + d1b_ref[...]
    d2 = d2_ref[...]
    g1 = g1_ref[...]
    g2 = g2_ref[...]
    g2d = g_ref[...].reshape(16 * t, 128)
    kv = lax.bitcast_convert_type(g2d[:, 0:64], jnp.int32)
    kk = lax.bitcast_convert_type(kv & jnp.int32(-65536), jnp.float32)
    vv = lax.bitcast_convert_type(kv << 16, jnp.float32)
    qs = jnp.broadcast_to(q[None], (16, t, 64)).reshape(16 * t, 64)
    pxqs = jnp.broadcast_to(pxq[None], (16, t, 64)).reshape(16 * t, 64)
    pe = _dotT(jnp.maximum(pxqs - g2d[:, 64:128], 0.0), d2) + d2b_ref[...]
    a = _dotT(jnp.maximum(_dotT(qs - kk + pe, g1) + g1b_ref[...],
                          0.0), g2) + g2b_ref[...]
    a = a * 0.125
    vpe = vv + pe
    attns = [a[j * t:(j + 1) * t] for j in range(16)]
    mx = attns[0]
    for aj in attns[1:]:
        mx = jnp.maximum(mx, aj)
    es = [jnp.exp(aj - mx) for aj in attns]
    tot = es[0]
    for e in es[1:]:
        tot = tot + e
    res = jnp.zeros_like(q)
    for j, e in enumerate(es):
        res = res + e * vpe[j * t:(j + 1) * t]
    res = res / tot
    o_ref[...] = _dotT(res, fc2_ref[...]) + fc2b_ref[...] + pre_ref[...]


def _tf_attn(gathered, qarr, px, pre, p, blk):
    m, dp = pre.shape
    grid = (m // blk,)
    full = lambda shape: pl.BlockSpec(shape, lambda n: (0,) * len(shape))
    return pl.pallas_call(
        _tf_attn_body,
        grid=grid,
        in_specs=[
            pl.BlockSpec((16, blk, 128), lambda n: (0, n, 0)),
            pl.BlockSpec((blk, 64), lambda n: (n, 0)),
            pl.BlockSpec((blk, 64), lambda n: (n, 0)),
            pl.BlockSpec((blk, dp), lambda n: (n, 0)),
            full((1, 64)), full((64, 64)), full((1, 64)),
            full((64, 64)), full((1, 64)), full((64, 64)), full((1, 64)),
            full((dp, 64)), full((1, dp)),
        ],
        out_specs=pl.BlockSpec((blk, dp), lambda n: (n, 0)),
        out_shape=jax.ShapeDtypeStruct((m, dp), jnp.float32),
    )(gathered, qarr, px, pre,
      p['d1_b'].reshape(1, -1), p['d2_w'], p['d2_b'].reshape(1, -1),
      p['g1_w'], p['g1_b'].reshape(1, -1), p['g2_w'], p['g2_b'].reshape(1, -1),
      p['fc2_w'], p['fc2_b'].reshape(1, -1))


def _transition_up(xyz_src, pts_src, xyz_q, pts_q, p, blk):
    bsz, n_src, d1 = pts_src.shape
    n_q, d2 = pts_q.shape[1], pts_q.shape[2]
    dout = p['fc1_w'].shape[0]
    f1 = _lin_bn_relu(pts_src.reshape(bsz * n_src, d1), p['fc1_w'], p['fc1_b'],
                      p['bn1_g'], p['bn1_b'])
    f2 = _lin_bn_relu(pts_q.reshape(bsz * n_q, d2), p['fc2_w'], p['fc2_b'],
                      p['bn2_g'], p['bn2_b'])
    src_T = jnp.transpose(xyz_src, (0, 2, 1))
    q_flat = xyz_q.reshape(bsz * n_q, 3)
    f1b = f1.reshape(bsz, n_src, dout)
    return _tu_interp(q_flat, src_T, f1b, f2, bsz, n_q, n_src, dout, blk)


def _transformer(xyz, feats_flat, p, blk_topk, blk_attn):
    bsz, n_pts = xyz.shape[:2]
    m = bsz * n_pts
    xyz_flat = xyz.reshape(m, 3)
    xyzT = jnp.transpose(xyz, (0, 2, 1))
    qarr, table = _tf_proj(feats_flat, xyz_flat, p)
    sc_gather = _make_sc_gather(16 * n_pts, 128)
    idxs = [_tf_topk(xyz_flat[b * n_pts:(b + 1) * n_pts], xyzT[b:b + 1],
                     n_pts, 16, blk_topk, b * n_pts) for b in range(bsz)]
    gathers = [sc_gather(table, jnp.transpose(i, (1, 0)).reshape(16 * n_pts))
               for i in idxs]
    outs = []
    for b in range(bsz):
        lo, hi = b * n_pts, (b + 1) * n_pts
        outs.append(_tf_attn(gathers[b].reshape(16, n_pts, 128),
                             qarr[lo:hi], lax.slice(table, (lo, 64), (hi, 128)),
                             feats_flat[lo:hi], p, blk_attn))
    return jnp.concatenate(outs, axis=0)


def kernel(xyz, points, xyz1, feats1_in, xyz0, feats0_in, params):
    bsz, n2 = xyz.shape[:2]
    n1, n0 = xyz1.shape[1], xyz0.shape[1]

    pts = _transition_up(xyz, points, xyz1, feats1_in, params['tu0'], blk=256)
    pts = _transformer(xyz1, pts, params['tf0'], blk_topk=256, blk_attn=128)
    pts3 = pts.reshape(bsz, n1, -1)
    pts = _transition_up(xyz1, pts3, xyz0, feats0_in, params['tu1'], blk=256)
    pts = _transformer(xyz0, pts, params['tf1'], blk_topk=256, blk_attn=128)
    return xyz0, pts.reshape(bsz, n0, -1)
